# baseline (speedup 1.0000x reference)
import functools

import jax
import jax.numpy as jnp
import numpy as np
from jax import lax
from jax.experimental import pallas as pl
from jax.experimental.pallas import tpu as pltpu
from jax.experimental.pallas import tpu_sc as plsc

_B, _S, _V = 32, 200, 1000
_NBI = _S - 1
_NTRI = _S - 2
_NCH = 13
_VP = 1024

_DUMP_BI = 1_000_000
_TBL_BI = 1_000_016
_DUMP_TRI = 6_368_000
_TBL_TRI = 6_368_016
_DUMP_PR = _B * _B * _NTRI
_TBL_PR = _DUMP_PR + 16

_mesh = plsc.VectorSubcoreMesh(core_axis_name="c", subcore_axis_name="s")


def _wid():
    return lax.axis_index("s") * 2 + lax.axis_index("c")


def _conf_body(lg_ref, out_ref):
    i = pl.program_id(0)
    x = lg_ref[...]
    m = jnp.max(x, axis=1, keepdims=True)
    s = jnp.sum(jnp.exp(x - m), axis=1)
    part = jnp.sum(1.0 / s)

    @pl.when(i == 0)
    def _():
        out_ref[...] = jnp.zeros((1, 1), jnp.float32)

    out_ref[...] += jnp.full((1, 1), part)


def _prep_body(toks_ref, bik_ref, t2k_ref):
    toks = toks_ref[...]
    bi = toks[:, :-1] * _V + toks[:, 1:]
    bik_ref[...] = jnp.concatenate(
        [bi, jnp.full((_B, 57), _DUMP_BI, jnp.int32)], axis=1)
    t2k_ref[...] = jnp.concatenate(
        [toks[:, 2:], jnp.zeros((_B, 58), jnp.int32)], axis=1)


def _stats_body(toks_ref, pbi_ref, ptri_ref, ppr_ref, conf_ref, out_ref,
                counts_ref, pres_ref):
    counts_ref[...] = jnp.zeros((1, _V), jnp.float32)
    iota_v = lax.broadcasted_iota(jnp.int32, (1, _V), 1)

    def hist_body(b, _):
        row = toks_ref[b, :]
        cmp = row[:, None] == iota_v
        counts_ref[...] += jnp.sum(cmp.astype(jnp.float32), axis=0)[None, :]
        pres_ref[pl.ds(b, 1), :] = jnp.any(cmp, axis=0).astype(jnp.float32)[None, :]
        return 0

    lax.fori_loop(0, _B, hist_body, 0)
    counts = counts_ref[0, :]
    total = jnp.sum(counts)
    probs = counts / (total + 1e-08)
    entropy = -jnp.sum(jnp.where(probs > 0, probs * jnp.log(probs + 1e-08), 0.0))
    token_entropy = 1.0 - entropy / np.log(_V)
    distinct1 = jnp.sum((counts > 0).astype(jnp.float32))

    pres = pres_ref[...]
    inter = lax.dot_general(pres, pres, (((1,), (1,)), ((), ())),
                            preferred_element_type=jnp.float32)
    ru = jnp.sum(pres, axis=1)
    r_i = lax.broadcasted_iota(jnp.int32, (_B, _B), 0)
    c_i = lax.broadcasted_iota(jnp.int32, (_B, _B), 1)
    selmask = ((r_i < 10) & (r_i != c_i)).astype(jnp.float32)
    overlaps = inter / jnp.maximum(ru, 1.0)[:, None]
    self_bleu = jnp.sum(overlaps * selmask) / (10 * (_B - 1))

    u_bi = jnp.sum(pbi_ref[...]).astype(jnp.float32)
    u_tri = jnp.sum(ptri_ref[...]).astype(jnp.float32)
    u_pr = jnp.sum(ppr_ref[...]).astype(jnp.float32)

    repetition = 1.0 - u_pr / (_B * _NTRI)
    d1 = distinct1 / (_B * _S)
    d2 = u_bi / (_B * _NBI)
    d3 = u_tri / (_B * _NTRI)
    ngram_diversity = ((1.0 - d1) + (1.0 - d2) + (1.0 - d3)) / 3.0

    avg_conf = jnp.sum(conf_ref[...]) / (_B * _S)
    overconfidence = jnp.maximum(avg_conf - 0.85, 0.0) * 2.0

    total_loss = (0.25 * ngram_diversity + 0.2 * token_entropy + 0.2 * self_bleu
                  + 0.2 * repetition + 0.15 * overconfidence)

    out_ref[...] = jnp.stack([ngram_diversity, token_entropy, self_bleu,
                              repetition, overconfidence, total_loss])[None, :]


@functools.partial(
    pl.kernel, mesh=_mesh,
    out_type=jax.ShapeDtypeStruct((_TBL_BI,), jnp.int32),
    scratch_types=[pltpu.VMEM((128,), jnp.int32),
                   pltpu.VMEM((128,), jnp.int32),
                   pltpu.VMEM((128,), jnp.int32),
                   pltpu.VMEM((128,), jnp.int32),
                   pltpu.SemaphoreType.DMA])
def _sc_bi_scatter(bik_hbm, tbl_hbm, key_a, key_b, val_a, val_b, sem):
    b = _wid()
    pltpu.sync_copy(bik_hbm.at[b].at[pl.ds(0, 128)], key_a)
    pltpu.sync_copy(bik_hbm.at[b].at[pl.ds(128, 128)], key_b)
    iota = lax.iota(jnp.int32, 16)
    for c in range(8):
        val_a[pl.ds(c * 16, 16)] = b * _NBI + c * 16 + iota
        val_b[pl.ds(c * 16, 16)] = b * _NBI + 128 + c * 16 + iota
    cp1 = pltpu.async_copy(val_a, tbl_hbm.at[key_a], sem)
    cp2 = pltpu.async_copy(val_b, tbl_hbm.at[key_b], sem)
    cp1.wait()
    cp2.wait()


@functools.partial(
    pl.kernel, mesh=_mesh,
    out_type=[jax.ShapeDtypeStruct((_TBL_TRI,), jnp.int32),
              jax.ShapeDtypeStruct((_B, 16), jnp.int32),
              jax.ShapeDtypeStruct((_B, 256), jnp.int32)],
    scratch_types=[pltpu.VMEM((128,), jnp.int32) for _ in range(10)]
                  + [pltpu.VMEM((16,), jnp.int32), pltpu.SemaphoreType.DMA])
def _sc_tri_scatter(bik_hbm, t2k_hbm, tblbi_hbm,
                    tbltri_hbm, part_hbm, keys3_hbm,
                    key_a, key_b, t2_a, t2_b, g_a, g_b,
                    k3_a, k3_b, v3_a, v3_b, acc_v, sem):
    b = _wid()
    pltpu.sync_copy(bik_hbm.at[b].at[pl.ds(0, 128)], key_a)
    pltpu.sync_copy(bik_hbm.at[b].at[pl.ds(128, 128)], key_b)
    pltpu.sync_copy(t2k_hbm.at[b].at[pl.ds(0, 128)], t2_a)
    pltpu.sync_copy(t2k_hbm.at[b].at[pl.ds(128, 128)], t2_b)
    cp1 = pltpu.async_copy(tblbi_hbm.at[key_a], g_a, sem)
    cp2 = pltpu.async_copy(tblbi_hbm.at[key_b], g_b, sem)
    cp1.wait()
    cp2.wait()
    iota = lax.iota(jnp.int32, 16)
    one = jnp.ones((16,), jnp.int32)
    zero = jnp.zeros((16,), jnp.int32)
    dump3 = jnp.full((16,), _DUMP_TRI, jnp.int32)
    acc = zero
    for half, (g_h, t2_h, k3_h, v3_h) in enumerate(
            [(g_a, t2_a, k3_a, v3_a), (g_b, t2_b, k3_b, v3_b)]):
        for c in range(8):
            s_c = half * 128 + c * 16 + iota
            sl = pl.ds(c * 16, 16)
            g_c = g_h[sl]
            j_c = b * _NBI + s_c
            acc = acc + jnp.where((s_c < _NBI) & (g_c == j_c), one, zero)
            k3_h[sl] = jnp.where(s_c < _NTRI, g_c * _V + t2_h[sl], dump3)
            v3_h[sl] = b * _NTRI + s_c
    acc_v[...] = acc
    pltpu.sync_copy(acc_v, part_hbm.at[b])
    pltpu.sync_copy(k3_a, keys3_hbm.at[b].at[pl.ds(0, 128)])
    pltpu.sync_copy(k3_b, keys3_hbm.at[b].at[pl.ds(128, 128)])
    cp1 = pltpu.async_copy(v3_a, tbltri_hbm.at[k3_a], sem)
    cp2 = pltpu.async_copy(v3_b, tbltri_hbm.at[k3_b], sem)
    cp1.wait()
    cp2.wait()


@functools.partial(
    pl.kernel, mesh=_mesh,
    out_type=[jax.ShapeDtypeStruct((_TBL_PR,), jnp.int32),
              jax.ShapeDtypeStruct((_B, 16), jnp.int32),
              jax.ShapeDtypeStruct((_B, 256), jnp.int32)],
    scratch_types=[pltpu.VMEM((128,), jnp.int32) for _ in range(8)]
                  + [pltpu.VMEM((16,), jnp.int32), pltpu.SemaphoreType.DMA])
def _sc_pr_scatter(keys3_hbm, tbltri_hbm,
                   tblpr_hbm, part_hbm, keys4_hbm,
                   k3_a, k3_b, h_a, h_b, k4_a, k4_b, v_a, v_b, acc_v, sem):
    b = _wid()
    pltpu.sync_copy(keys3_hbm.at[b].at[pl.ds(0, 128)], k3_a)
    pltpu.sync_copy(keys3_hbm.at[b].at[pl.ds(128, 128)], k3_b)
    cp1 = pltpu.async_copy(tbltri_hbm.at[k3_a], h_a, sem)
    cp2 = pltpu.async_copy(tbltri_hbm.at[k3_b], h_b, sem)
    cp1.wait()
    cp2.wait()
    iota = lax.iota(jnp.int32, 16)
    one = jnp.ones((16,), jnp.int32)
    zero = jnp.zeros((16,), jnp.int32)
    dump4 = jnp.full((16,), _DUMP_PR, jnp.int32)
    acc = zero
    for half, (h_h, k4_h, v_h) in enumerate(
            [(h_a, k4_a, v_a), (h_b, k4_b, v_b)]):
        for c in range(8):
            s_c = half * 128 + c * 16 + iota
            sl = pl.ds(c * 16, 16)
            h_c = h_h[sl]
            j3_c = b * _NTRI + s_c
            valid = s_c < _NTRI
            acc = acc + jnp.where(valid & (h_c == j3_c), one, zero)
            k4_h[sl] = jnp.where(valid, b * (_B * _NTRI) + h_c, dump4)
            v_h[sl] = j3_c
    acc_v[...] = acc
    pltpu.sync_copy(acc_v, part_hbm.at[b])
    pltpu.sync_copy(k4_a, keys4_hbm.at[b].at[pl.ds(0, 128)])
    pltpu.sync_copy(k4_b, keys4_hbm.at[b].at[pl.ds(128, 128)])
    cp1 = pltpu.async_copy(v_a, tblpr_hbm.at[k4_a], sem)
    cp2 = pltpu.async_copy(v_b, tblpr_hbm.at[k4_b], sem)
    cp1.wait()
    cp2.wait()


@functools.partial(
    pl.kernel, mesh=_mesh,
    out_type=jax.ShapeDtypeStruct((_B, 16), jnp.int32),
    scratch_types=[pltpu.VMEM((128,), jnp.int32) for _ in range(4)]
                  + [pltpu.VMEM((16,), jnp.int32), pltpu.SemaphoreType.DMA])
def _sc_pr_gather(keys4_hbm, tblpr_hbm, part_hbm,
                  k4_a, k4_b, p_a, p_b, acc_v, sem):
    b = _wid()
    pltpu.sync_copy(keys4_hbm.at[b].at[pl.ds(0, 128)], k4_a)
    pltpu.sync_copy(keys4_hbm.at[b].at[pl.ds(128, 128)], k4_b)
    cp1 = pltpu.async_copy(tblpr_hbm.at[k4_a], p_a, sem)
    cp2 = pltpu.async_copy(tblpr_hbm.at[k4_b], p_b, sem)
    cp1.wait()
    cp2.wait()
    iota = lax.iota(jnp.int32, 16)
    one = jnp.ones((16,), jnp.int32)
    zero = jnp.zeros((16,), jnp.int32)
    acc = zero
    for half, p_h in enumerate([p_a, p_b]):
        for c in range(8):
            s_c = half * 128 + c * 16 + iota
            j3_c = b * _NTRI + s_c
            acc = acc + jnp.where((s_c < _NTRI) & (p_h[pl.ds(c * 16, 16)] == j3_c),
                                  one, zero)
    acc_v[...] = acc
    pltpu.sync_copy(acc_v, part_hbm.at[b])


@jax.jit
def _run(toks, logits):
    toks = toks.astype(jnp.int32)
    lg2 = logits.reshape(_B * _S, _V)
    rows = 800
    conf = pl.pallas_call(
        _conf_body,
        grid=(_B * _S // rows,),
        in_specs=[pl.BlockSpec((rows, _V), lambda i: (i, 0))],
        out_specs=pl.BlockSpec((1, 1), lambda i: (0, 0)),
        out_shape=jax.ShapeDtypeStruct((1, 1), jnp.float32),
    )(lg2)

    bik, t2k = pl.pallas_call(
        _prep_body,
        out_shape=[jax.ShapeDtypeStruct((_B, 256), jnp.int32),
                   jax.ShapeDtypeStruct((_B, 256), jnp.int32)],
    )(toks)

    tbl_bi = _sc_bi_scatter(bik)
    tbl_tri, part_bi, keys3 = _sc_tri_scatter(bik, t2k, tbl_bi)
    tbl_pr, part_tri, keys4 = _sc_pr_scatter(keys3, tbl_tri)
    part_pr = _sc_pr_gather(keys4, tbl_pr)

    out = pl.pallas_call(
        _stats_body,
        out_shape=jax.ShapeDtypeStruct((1, 6), jnp.float32),
        scratch_shapes=[pltpu.VMEM((1, _V), jnp.float32),
                        pltpu.VMEM((_B, _V), jnp.float32)],
    )(toks, part_bi, part_tri, part_pr, conf)
    return out.reshape(6)


def kernel(generated_tokens, generated_logits, vocab_size):
    return _run(generated_tokens, generated_logits)


# speedup vs baseline: 22.9194x; 22.9194x over previous
import functools

import jax
import jax.numpy as jnp
import numpy as np
from jax import lax
from jax.experimental import pallas as pl
from jax.experimental.pallas import tpu as pltpu
from jax.experimental.pallas import tpu_sc as plsc

_B, _S, _V = 32, 200, 1000
_NBI = _S - 1
_NTRI = _S - 2
_NCH = 13

_DUMP = 1_040_000
_SH_WORDS = 1_048_576

_mesh = plsc.VectorSubcoreMesh(core_axis_name="c", subcore_axis_name="s",
                               num_cores=1)


def _conf_body(lg_ref, out_ref):
    i = pl.program_id(0)
    x = lg_ref[...]
    m = jnp.max(x, axis=1, keepdims=True)
    s = jnp.sum(jnp.exp(x - m), axis=1)
    part = jnp.sum(1.0 / s)

    @pl.when(i == 0)
    def _():
        out_ref[...] = jnp.zeros((1, 1), jnp.float32)

    out_ref[...] += jnp.full((1, 1), part)


def _prep_body(toks_ref, bik_ref, t2k_ref):
    toks = toks_ref[...]
    bi = toks[:, :-1] * _V + toks[:, 1:]
    bik_ref[...] = jnp.concatenate(
        [bi, jnp.full((_B, 9), _DUMP, jnp.int32)], axis=1)
    t2k_ref[...] = jnp.concatenate(
        [toks[:, 2:], jnp.zeros((_B, 10), jnp.int32)], axis=1)


def _stats_body(toks_ref, pbi_ref, ptri_ref, ppr_ref, conf_ref, out_ref,
                counts_ref, pres_ref):
    counts_ref[...] = jnp.zeros((1, _V), jnp.float32)
    iota_v = lax.broadcasted_iota(jnp.int32, (1, _V), 1)

    def hist_body(b, _):
        row = toks_ref[b, :]
        cmp = row[:, None] == iota_v
        counts_ref[...] += jnp.sum(cmp.astype(jnp.float32), axis=0)[None, :]
        pres_ref[pl.ds(b, 1), :] = jnp.any(cmp, axis=0).astype(jnp.float32)[None, :]
        return 0

    lax.fori_loop(0, _B, hist_body, 0)
    counts = counts_ref[0, :]
    total = jnp.sum(counts)
    probs = counts / (total + 1e-08)
    entropy = -jnp.sum(jnp.where(probs > 0, probs * jnp.log(probs + 1e-08), 0.0))
    token_entropy = 1.0 - entropy / np.log(_V)
    distinct1 = jnp.sum((counts > 0).astype(jnp.float32))

    pres = pres_ref[...]
    inter = lax.dot_general(pres, pres, (((1,), (1,)), ((), ())),
                            preferred_element_type=jnp.float32)
    ru = jnp.sum(pres, axis=1)
    r_i = lax.broadcasted_iota(jnp.int32, (_B, _B), 0)
    c_i = lax.broadcasted_iota(jnp.int32, (_B, _B), 1)
    selmask = ((r_i < 10) & (r_i != c_i)).astype(jnp.float32)
    overlaps = inter / jnp.maximum(ru, 1.0)[:, None]
    self_bleu = jnp.sum(overlaps * selmask) / (10 * (_B - 1))

    u_bi = jnp.sum(pbi_ref[...]).astype(jnp.float32)
    u_tri = jnp.sum(ptri_ref[...]).astype(jnp.float32)
    u_pr = jnp.sum(ppr_ref[...]).astype(jnp.float32)

    repetition = 1.0 - u_pr / (_B * _NTRI)
    d1 = distinct1 / (_B * _S)
    d2 = u_bi / (_B * _NBI)
    d3 = u_tri / (_B * _NTRI)
    ngram_diversity = ((1.0 - d1) + (1.0 - d2) + (1.0 - d3)) / 3.0

    avg_conf = jnp.sum(conf_ref[...]) / (_B * _S)
    overconfidence = jnp.maximum(avg_conf - 0.85, 0.0) * 2.0

    total_loss = (0.25 * ngram_diversity + 0.2 * token_entropy + 0.2 * self_bleu
                  + 0.2 * repetition + 0.15 * overconfidence)

    out_ref[...] = jnp.stack([ngram_diversity, token_entropy, self_bleu,
                              repetition, overconfidence, total_loss])[None, :]


def _v16(x):
    return jnp.full((16,), x, jnp.int32)


@functools.partial(
    pl.kernel, mesh=_mesh,
    out_type=[jax.ShapeDtypeStruct((_B, 16), jnp.int32),
              jax.ShapeDtypeStruct((_B, 16), jnp.int32),
              jax.ShapeDtypeStruct((_B, 16), jnp.int32)],
    scratch_types=[pltpu.VMEM_SHARED((_SH_WORDS,), jnp.int32)]
                  + [pltpu.VMEM((_NCH, 16), jnp.int32) for _ in range(10)]
                  + [pltpu.VMEM((16,), jnp.int32), pltpu.SemaphoreType.DMA])
def _sc_distinct(bik_hbm, t2k_hbm, pbi_hbm, ptri_hbm, ppr_hbm,
                 shared,
                 key0, key1, t20, t21, ki0, ki1,
                 v0, v1, g0, g1, acc_v, sem):
    w = lax.axis_index("s")
    rows = (2 * w, 2 * w + 1)
    keys = (key0, key1)
    t2s = (t20, t21)
    kis = (ki0, ki1)
    vs = (v0, v1)
    gs = (g0, g1)
    iota = lax.iota(jnp.int32, 16)
    one = jnp.ones((16,), jnp.int32)
    zero = jnp.zeros((16,), jnp.int32)

    def scatter_all():
        cps = []
        for i in (0, 1):
            for c in range(_NCH):
                cps.append(pltpu.async_copy(vs[i].at[c],
                                            shared.at[kis[i].at[c]], sem))
        for cp in cps:
            cp.wait()
        plsc.subcore_barrier()

    def gather_all():
        cps = []
        for i in (0, 1):
            for c in range(_NCH):
                cps.append(pltpu.async_copy(shared.at[kis[i].at[c]],
                                            gs[i].at[c], sem))
        for cp in cps:
            cp.wait()

    for i in (0, 1):
        pltpu.sync_copy(bik_hbm.at[rows[i]], keys[i])
        pltpu.sync_copy(t2k_hbm.at[rows[i]], t2s[i])

    for i in (0, 1):
        for c in range(_NCH):
            kis[i][c, :] = keys[i][c, :]
            vs[i][c, :] = rows[i] * _NBI + c * 16 + iota
    scatter_all()
    gather_all()
    for i in (0, 1):
        acc = zero
        for c in range(_NCH):
            s_c = c * 16 + iota
            g_c = gs[i][c, :]
            j_c = rows[i] * _NBI + s_c
            acc = acc + jnp.where((s_c < _NBI) & (g_c == j_c), one, zero)
            kis[i][c, :] = jnp.where(s_c < _NTRI,
                                     g_c * 8 + (t2s[i][c, :] >> 7), _v16(_DUMP))
            vs[i][c, :] = rows[i] * _NTRI + s_c
        acc_v[...] = acc
        pltpu.sync_copy(acc_v, pbi_hbm.at[rows[i]])
    plsc.subcore_barrier()

    scatter_all()
    gather_all()
    for i in (0, 1):
        for c in range(_NCH):
            s_c = c * 16 + iota
            q_c = gs[i][c, :]
            kis[i][c, :] = jnp.where(s_c < _NTRI,
                                     q_c * 128 + (t2s[i][c, :] & 127), _v16(_DUMP))
    plsc.subcore_barrier()

    scatter_all()
    gather_all()
    for i in (0, 1):
        acc = zero
        for c in range(_NCH):
            s_c = c * 16 + iota
            h_c = gs[i][c, :]
            j3_c = rows[i] * _NTRI + s_c
            acc = acc + jnp.where((s_c < _NTRI) & (h_c == j3_c), one, zero)
            kis[i][c, :] = jnp.where(s_c < _NTRI,
                                     rows[i] * (_B * _NTRI) + h_c, _v16(_DUMP))
        acc_v[...] = acc
        pltpu.sync_copy(acc_v, ptri_hbm.at[rows[i]])
    plsc.subcore_barrier()

    scatter_all()
    gather_all()
    for i in (0, 1):
        acc = zero
        for c in range(_NCH):
            s_c = c * 16 + iota
            j3_c = rows[i] * _NTRI + s_c
            acc = acc + jnp.where((s_c < _NTRI) & (gs[i][c, :] == j3_c), one, zero)
        acc_v[...] = acc
        pltpu.sync_copy(acc_v, ppr_hbm.at[rows[i]])


@jax.jit
def _run(toks, logits):
    toks = toks.astype(jnp.int32)
    lg2 = logits.reshape(_B * _S, _V)
    rows = 800
    conf = pl.pallas_call(
        _conf_body,
        grid=(_B * _S // rows,),
        in_specs=[pl.BlockSpec((rows, _V), lambda i: (i, 0))],
        out_specs=pl.BlockSpec((1, 1), lambda i: (0, 0)),
        out_shape=jax.ShapeDtypeStruct((1, 1), jnp.float32),
    )(lg2)

    bik, t2k = pl.pallas_call(
        _prep_body,
        out_shape=[jax.ShapeDtypeStruct((_B, 208), jnp.int32),
                   jax.ShapeDtypeStruct((_B, 208), jnp.int32)],
    )(toks)
    bik3 = bik.reshape(_B, _NCH, 16)
    t2k3 = t2k.reshape(_B, _NCH, 16)

    part_bi, part_tri, part_pr = _sc_distinct(bik3, t2k3)

    out = pl.pallas_call(
        _stats_body,
        out_shape=jax.ShapeDtypeStruct((1, 6), jnp.float32),
        scratch_shapes=[pltpu.VMEM((1, _V), jnp.float32),
                        pltpu.VMEM((_B, _V), jnp.float32)],
    )(toks, part_bi, part_tri, part_pr, conf)
    return out.reshape(6)


def kernel(generated_tokens, generated_logits, vocab_size):
    return _run(generated_tokens, generated_logits)


# speedup vs baseline: 23.8367x; 1.0400x over previous
import functools

import jax
import jax.numpy as jnp
import numpy as np
from jax import lax
from jax.experimental import pallas as pl
from jax.experimental.pallas import tpu as pltpu
from jax.experimental.pallas import tpu_sc as plsc

_B, _S, _V = 32, 200, 1000
_NBI = _S - 1
_NTRI = _S - 2
_NCH = 13

_DUMP = 1_040_000
_SH_WORDS = 1_048_576

_mesh = plsc.VectorSubcoreMesh(core_axis_name="c", subcore_axis_name="s",
                               num_cores=1)


def _conf_hist_body(lg_ref, toks_ref, conf_ref, counts_ref, pres_ref):
    i = pl.program_id(0)
    x = lg_ref[...]
    m = jnp.max(x, axis=1, keepdims=True)
    s = jnp.sum(jnp.exp(x - m), axis=1)
    part = jnp.sum(1.0 / s)

    tok4 = toks_ref[0]
    iota_v = lax.broadcasted_iota(jnp.int32, (1, 1, _V), 2)
    cmp = tok4[:, :, None] == iota_v
    cnt = jnp.sum(cmp.astype(jnp.float32), axis=(0, 1))
    pres_ref[...] = jnp.any(cmp, axis=1).astype(jnp.float32)[None]

    @pl.when(i == 0)
    def _():
        conf_ref[...] = jnp.zeros((1, 1), jnp.float32)
        counts_ref[...] = jnp.zeros((1, _V), jnp.float32)

    conf_ref[...] += jnp.full((1, 1), part)
    counts_ref[...] += cnt[None, :]


def _prep_body(toks_ref, bik_ref, t2k_ref):
    toks = toks_ref[...]
    bi = toks[:, :-1] * _V + toks[:, 1:]
    bik_ref[...] = jnp.concatenate(
        [bi, jnp.full((_B, 9), _DUMP, jnp.int32)], axis=1)
    t2k_ref[...] = jnp.concatenate(
        [toks[:, 2:], jnp.zeros((_B, 10), jnp.int32)], axis=1)


def _stats_body(counts_ref, pres_ref, pbi_ref, ptri_ref, ppr_ref, conf_ref,
                out_ref):
    counts = counts_ref[0, :]
    total = jnp.sum(counts)
    probs = counts / (total + 1e-08)
    entropy = -jnp.sum(jnp.where(probs > 0, probs * jnp.log(probs + 1e-08), 0.0))
    token_entropy = 1.0 - entropy / np.log(_V)
    distinct1 = jnp.sum((counts > 0).astype(jnp.float32))

    pres = pres_ref[...]
    inter = lax.dot_general(pres, pres, (((1,), (1,)), ((), ())),
                            preferred_element_type=jnp.float32)
    ru = jnp.sum(pres, axis=1)
    r_i = lax.broadcasted_iota(jnp.int32, (_B, _B), 0)
    c_i = lax.broadcasted_iota(jnp.int32, (_B, _B), 1)
    selmask = ((r_i < 10) & (r_i != c_i)).astype(jnp.float32)
    overlaps = inter / jnp.maximum(ru, 1.0)[:, None]
    self_bleu = jnp.sum(overlaps * selmask) / (10 * (_B - 1))

    u_bi = jnp.sum(pbi_ref[...]).astype(jnp.float32)
    u_tri = jnp.sum(ptri_ref[...]).astype(jnp.float32)
    u_pr = jnp.sum(ppr_ref[...]).astype(jnp.float32)

    repetition = 1.0 - u_pr / (_B * _NTRI)
    d1 = distinct1 / (_B * _S)
    d2 = u_bi / (_B * _NBI)
    d3 = u_tri / (_B * _NTRI)
    ngram_diversity = ((1.0 - d1) + (1.0 - d2) + (1.0 - d3)) / 3.0

    avg_conf = jnp.sum(conf_ref[...]) / (_B * _S)
    overconfidence = jnp.maximum(avg_conf - 0.85, 0.0) * 2.0

    total_loss = (0.25 * ngram_diversity + 0.2 * token_entropy + 0.2 * self_bleu
                  + 0.2 * repetition + 0.15 * overconfidence)

    out_ref[...] = jnp.stack([ngram_diversity, token_entropy, self_bleu,
                              repetition, overconfidence, total_loss])[None, :]


def _v16(x):
    return jnp.full((16,), x, jnp.int32)


@functools.partial(
    pl.kernel, mesh=_mesh,
    out_type=[jax.ShapeDtypeStruct((_B, 16), jnp.int32),
              jax.ShapeDtypeStruct((_B, 16), jnp.int32),
              jax.ShapeDtypeStruct((_B, 16), jnp.int32)],
    scratch_types=[pltpu.VMEM_SHARED((_SH_WORDS,), jnp.int32)]
                  + [pltpu.VMEM((_NCH, 16), jnp.int32) for _ in range(10)]
                  + [pltpu.VMEM((16,), jnp.int32), pltpu.SemaphoreType.DMA])
def _sc_distinct(bik_hbm, t2k_hbm, pbi_hbm, ptri_hbm, ppr_hbm,
                 shared,
                 key0, key1, t20, t21, ki0, ki1,
                 v0, v1, g0, g1, acc_v, sem):
    w = lax.axis_index("s")
    rows = (2 * w, 2 * w + 1)
    keys = (key0, key1)
    t2s = (t20, t21)
    kis = (ki0, ki1)
    vs = (v0, v1)
    gs = (g0, g1)
    iota = lax.iota(jnp.int32, 16)
    one = jnp.ones((16,), jnp.int32)
    zero = jnp.zeros((16,), jnp.int32)

    def scatter_all():
        cps = []
        for i in (0, 1):
            for c in range(_NCH):
                cps.append(pltpu.async_copy(vs[i].at[c],
                                            shared.at[kis[i].at[c]], sem))
        for cp in cps:
            cp.wait()
        plsc.subcore_barrier()

    def gather_all():
        cps = []
        for i in (0, 1):
            for c in range(_NCH):
                cps.append(pltpu.async_copy(shared.at[kis[i].at[c]],
                                            gs[i].at[c], sem))
        for cp in cps:
            cp.wait()

    for i in (0, 1):
        pltpu.sync_copy(bik_hbm.at[rows[i]], keys[i])
        pltpu.sync_copy(t2k_hbm.at[rows[i]], t2s[i])

    for i in (0, 1):
        for c in range(_NCH):
            kis[i][c, :] = keys[i][c, :]
            vs[i][c, :] = rows[i] * _NBI + c * 16 + iota
    scatter_all()
    gather_all()
    for i in (0, 1):
        acc = zero
        for c in range(_NCH):
            s_c = c * 16 + iota
            g_c = gs[i][c, :]
            j_c = rows[i] * _NBI + s_c
            acc = acc + jnp.where((s_c < _NBI) & (g_c == j_c), one, zero)
            kis[i][c, :] = jnp.where(s_c < _NTRI,
                                     g_c * 8 + (t2s[i][c, :] >> 7), _v16(_DUMP))
            vs[i][c, :] = rows[i] * _NTRI + s_c
        acc_v[...] = acc
        pltpu.sync_copy(acc_v, pbi_hbm.at[rows[i]])
    plsc.subcore_barrier()

    scatter_all()
    gather_all()
    for i in (0, 1):
        for c in range(_NCH):
            s_c = c * 16 + iota
            q_c = gs[i][c, :]
            kis[i][c, :] = jnp.where(s_c < _NTRI,
                                     q_c * 128 + (t2s[i][c, :] & 127), _v16(_DUMP))
    plsc.subcore_barrier()

    scatter_all()
    gather_all()
    for i in (0, 1):
        acc = zero
        for c in range(_NCH):
            s_c = c * 16 + iota
            h_c = gs[i][c, :]
            j3_c = rows[i] * _NTRI + s_c
            acc = acc + jnp.where((s_c < _NTRI) & (h_c == j3_c), one, zero)
            kis[i][c, :] = jnp.where(s_c < _NTRI,
                                     rows[i] * (_B * _NTRI) + h_c, _v16(_DUMP))
        acc_v[...] = acc
        pltpu.sync_copy(acc_v, ptri_hbm.at[rows[i]])
    plsc.subcore_barrier()

    scatter_all()
    gather_all()
    for i in (0, 1):
        acc = zero
        for c in range(_NCH):
            s_c = c * 16 + iota
            j3_c = rows[i] * _NTRI + s_c
            acc = acc + jnp.where((s_c < _NTRI) & (gs[i][c, :] == j3_c), one, zero)
        acc_v[...] = acc
        pltpu.sync_copy(acc_v, ppr_hbm.at[rows[i]])


@jax.jit
def _run(toks, logits):
    toks = toks.astype(jnp.int32)
    lg2 = logits.reshape(_B * _S, _V)
    rows = 800

    bik, t2k = pl.pallas_call(
        _prep_body,
        out_shape=[jax.ShapeDtypeStruct((_B, 208), jnp.int32),
                   jax.ShapeDtypeStruct((_B, 208), jnp.int32)],
    )(toks)
    bik3 = bik.reshape(_B, _NCH, 16)
    t2k3 = t2k.reshape(_B, _NCH, 16)

    part_bi, part_tri, part_pr = _sc_distinct(bik3, t2k3)

    conf, counts, pres = pl.pallas_call(
        _conf_hist_body,
        grid=(_B * _S // rows,),
        in_specs=[pl.BlockSpec((rows, _V), lambda i: (i, 0)),
                  pl.BlockSpec((1, 4, _S), lambda i: (i, 0, 0))],
        out_specs=[pl.BlockSpec((1, 1), lambda i: (0, 0)),
                   pl.BlockSpec((1, _V), lambda i: (0, 0)),
                   pl.BlockSpec((1, 4, _V), lambda i: (i, 0, 0))],
        out_shape=[jax.ShapeDtypeStruct((1, 1), jnp.float32),
                   jax.ShapeDtypeStruct((1, _V), jnp.float32),
                   jax.ShapeDtypeStruct((8, 4, _V), jnp.float32)],
    )(lg2, toks.reshape(8, 4, _S))
    pres = pres.reshape(_B, _V)

    out = pl.pallas_call(
        _stats_body,
        out_shape=jax.ShapeDtypeStruct((1, 6), jnp.float32),
    )(counts, pres, part_bi, part_tri, part_pr, conf)
    return out.reshape(6)


def kernel(generated_tokens, generated_logits, vocab_size):
    return _run(generated_tokens, generated_logits)


# speedup vs baseline: 26.7647x; 1.1228x over previous
import functools

import jax
import jax.numpy as jnp
import numpy as np
from jax import lax
from jax.experimental import pallas as pl
from jax.experimental.pallas import tpu as pltpu
from jax.experimental.pallas import tpu_sc as plsc

_B, _S, _V = 32, 200, 1000
_NBI = _S - 1
_NTRI = _S - 2
_NCH = 13

_DUMP = 1_040_000
_SH_WORDS = 1_048_576

_mesh = plsc.VectorSubcoreMesh(core_axis_name="c", subcore_axis_name="s",
                               num_cores=1)


def _conf_hist_body(lg_ref, toks_ref, conf_ref, counts_ref, pres_ref):
    i = pl.program_id(0)
    x = lg_ref[...]
    m = jnp.max(x, axis=1, keepdims=True)
    s = jnp.sum(jnp.exp(x - m), axis=1)
    part = jnp.sum(1.0 / s)

    tok8 = toks_ref[pl.ds(i * 8, 8), :]
    iota_v = lax.broadcasted_iota(jnp.int32, (1, 1, _V), 2)
    cmp = tok8[:, :, None] == iota_v
    cnt = jnp.sum(cmp.astype(jnp.float32), axis=(0, 1))
    pres_ref[pl.ds(i * 8, 8), :] = jnp.any(cmp, axis=1).astype(jnp.float32)

    @pl.when(i == 0)
    def _():
        conf_ref[...] = jnp.zeros((1, 1), jnp.float32)
        counts_ref[...] = jnp.zeros((1, _V), jnp.float32)

    conf_ref[...] += jnp.full((1, 1), part)
    counts_ref[...] += cnt[None, :]


def _prep_body(toks_ref, bik_ref, t2k_ref):
    toks = toks_ref[...]
    bi = toks[:, :-1] * _V + toks[:, 1:]
    bik_ref[...] = jnp.concatenate(
        [bi, jnp.full((_B, 9), _DUMP, jnp.int32)], axis=1)
    t2k_ref[...] = jnp.concatenate(
        [toks[:, 2:], jnp.zeros((_B, 10), jnp.int32)], axis=1)


def _stats_body(counts_ref, pres_ref, pbi_ref, ptri_ref, ppr_ref, conf_ref,
                out_ref):
    counts = counts_ref[0, :]
    total = jnp.sum(counts)
    probs = counts / (total + 1e-08)
    entropy = -jnp.sum(jnp.where(probs > 0, probs * jnp.log(probs + 1e-08), 0.0))
    token_entropy = 1.0 - entropy / np.log(_V)
    distinct1 = jnp.sum((counts > 0).astype(jnp.float32))

    pres = pres_ref[...]
    inter = lax.dot_general(pres, pres, (((1,), (1,)), ((), ())),
                            preferred_element_type=jnp.float32)
    ru = jnp.sum(pres, axis=1)
    r_i = lax.broadcasted_iota(jnp.int32, (_B, _B), 0)
    c_i = lax.broadcasted_iota(jnp.int32, (_B, _B), 1)
    selmask = ((r_i < 10) & (r_i != c_i)).astype(jnp.float32)
    overlaps = inter / jnp.maximum(ru, 1.0)[:, None]
    self_bleu = jnp.sum(overlaps * selmask) / (10 * (_B - 1))

    u_bi = jnp.sum(pbi_ref[...]).astype(jnp.float32)
    u_tri = jnp.sum(ptri_ref[...]).astype(jnp.float32)
    u_pr = jnp.sum(ppr_ref[...]).astype(jnp.float32)

    repetition = 1.0 - u_pr / (_B * _NTRI)
    d1 = distinct1 / (_B * _S)
    d2 = u_bi / (_B * _NBI)
    d3 = u_tri / (_B * _NTRI)
    ngram_diversity = ((1.0 - d1) + (1.0 - d2) + (1.0 - d3)) / 3.0

    avg_conf = jnp.sum(conf_ref[...]) / (_B * _S)
    overconfidence = jnp.maximum(avg_conf - 0.85, 0.0) * 2.0

    total_loss = (0.25 * ngram_diversity + 0.2 * token_entropy + 0.2 * self_bleu
                  + 0.2 * repetition + 0.15 * overconfidence)

    out_ref[...] = jnp.stack([ngram_diversity, token_entropy, self_bleu,
                              repetition, overconfidence, total_loss])[None, :]


def _v16(x):
    return jnp.full((16,), x, jnp.int32)


@functools.partial(
    pl.kernel, mesh=_mesh,
    out_type=[jax.ShapeDtypeStruct((_B, 16), jnp.int32),
              jax.ShapeDtypeStruct((_B, 16), jnp.int32),
              jax.ShapeDtypeStruct((_B, 16), jnp.int32)],
    scratch_types=[pltpu.VMEM_SHARED((_SH_WORDS,), jnp.int32)]
                  + [pltpu.VMEM((_NCH, 16), jnp.int32) for _ in range(10)]
                  + [pltpu.VMEM((16,), jnp.int32), pltpu.SemaphoreType.DMA])
def _sc_distinct(bik_hbm, t2k_hbm, pbi_hbm, ptri_hbm, ppr_hbm,
                 shared,
                 key0, key1, t20, t21, ki0, ki1,
                 v0, v1, g0, g1, acc_v, sem):
    w = lax.axis_index("s")
    rows = (2 * w, 2 * w + 1)
    keys = (key0, key1)
    t2s = (t20, t21)
    kis = (ki0, ki1)
    vs = (v0, v1)
    gs = (g0, g1)
    iota = lax.iota(jnp.int32, 16)
    one = jnp.ones((16,), jnp.int32)
    zero = jnp.zeros((16,), jnp.int32)

    def scatter_all():
        cps = []
        for i in (0, 1):
            for c in range(_NCH):
                cps.append(pltpu.async_copy(vs[i].at[c],
                                            shared.at[kis[i].at[c]], sem))
        for cp in cps:
            cp.wait()
        plsc.subcore_barrier()

    def gather_all():
        cps = []
        for i in (0, 1):
            for c in range(_NCH):
                cps.append(pltpu.async_copy(shared.at[kis[i].at[c]],
                                            gs[i].at[c], sem))
        for cp in cps:
            cp.wait()

    for i in (0, 1):
        pltpu.sync_copy(bik_hbm.at[rows[i]], keys[i])
        pltpu.sync_copy(t2k_hbm.at[rows[i]], t2s[i])

    for i in (0, 1):
        for c in range(_NCH):
            kis[i][c, :] = keys[i][c, :]
            vs[i][c, :] = rows[i] * _NBI + c * 16 + iota
    scatter_all()
    gather_all()
    for i in (0, 1):
        acc = zero
        for c in range(_NCH):
            s_c = c * 16 + iota
            g_c = gs[i][c, :]
            j_c = rows[i] * _NBI + s_c
            acc = acc + jnp.where((s_c < _NBI) & (g_c == j_c), one, zero)
            kis[i][c, :] = jnp.where(s_c < _NTRI,
                                     g_c * 8 + (t2s[i][c, :] >> 7), _v16(_DUMP))
            vs[i][c, :] = rows[i] * _NTRI + s_c
        acc_v[...] = acc
        pltpu.sync_copy(acc_v, pbi_hbm.at[rows[i]])
    plsc.subcore_barrier()

    scatter_all()
    gather_all()
    for i in (0, 1):
        for c in range(_NCH):
            s_c = c * 16 + iota
            q_c = gs[i][c, :]
            kis[i][c, :] = jnp.where(s_c < _NTRI,
                                     q_c * 128 + (t2s[i][c, :] & 127), _v16(_DUMP))
    plsc.subcore_barrier()

    scatter_all()
    gather_all()
    for i in (0, 1):
        acc = zero
        for c in range(_NCH):
            s_c = c * 16 + iota
            h_c = gs[i][c, :]
            j3_c = rows[i] * _NTRI + s_c
            acc = acc + jnp.where((s_c < _NTRI) & (h_c == j3_c), one, zero)
            kis[i][c, :] = jnp.where(s_c < _NTRI,
                                     rows[i] * (_B * _NTRI) + h_c, _v16(_DUMP))
        acc_v[...] = acc
        pltpu.sync_copy(acc_v, ptri_hbm.at[rows[i]])
    plsc.subcore_barrier()

    scatter_all()
    gather_all()
    for i in (0, 1):
        acc = zero
        for c in range(_NCH):
            s_c = c * 16 + iota
            j3_c = rows[i] * _NTRI + s_c
            acc = acc + jnp.where((s_c < _NTRI) & (gs[i][c, :] == j3_c), one, zero)
        acc_v[...] = acc
        pltpu.sync_copy(acc_v, ppr_hbm.at[rows[i]])


@jax.jit
def _run(toks, logits):
    toks = toks.astype(jnp.int32)
    lg2 = logits.reshape(_B * _S, _V)
    rows = 1600

    bik, t2k = pl.pallas_call(
        _prep_body,
        out_shape=[jax.ShapeDtypeStruct((_B, 208), jnp.int32),
                   jax.ShapeDtypeStruct((_B, 208), jnp.int32)],
    )(toks)
    bik3 = bik.reshape(_B, _NCH, 16)
    t2k3 = t2k.reshape(_B, _NCH, 16)

    part_bi, part_tri, part_pr = _sc_distinct(bik3, t2k3)

    conf, counts, pres = pl.pallas_call(
        _conf_hist_body,
        grid=(_B * _S // rows,),
        in_specs=[pl.BlockSpec((rows, _V), lambda i: (i, 0)),
                  pl.BlockSpec((_B, _S), lambda i: (0, 0))],
        out_specs=[pl.BlockSpec((1, 1), lambda i: (0, 0)),
                   pl.BlockSpec((1, _V), lambda i: (0, 0)),
                   pl.BlockSpec((_B, _V), lambda i: (0, 0))],
        out_shape=[jax.ShapeDtypeStruct((1, 1), jnp.float32),
                   jax.ShapeDtypeStruct((1, _V), jnp.float32),
                   jax.ShapeDtypeStruct((_B, _V), jnp.float32)],
    )(lg2, toks)

    out = pl.pallas_call(
        _stats_body,
        out_shape=jax.ShapeDtypeStruct((1, 6), jnp.float32),
    )(counts, pres, part_bi, part_tri, part_pr, conf)
    return out.reshape(6)


def kernel(generated_tokens, generated_logits, vocab_size):
    return _run(generated_tokens, generated_logits)
